# Initial kernel scaffold; baseline (speedup 1.0000x reference)
import jax
import jax.numpy as jnp
from jax.experimental import pallas as pl


def kernel(x, observed):
    raise NotImplementedError("write your pallas kernel here")


# speedup vs baseline: 2.9534x; 2.9534x over previous
import functools

import jax
import jax.numpy as jnp
from jax import lax
from jax.experimental import pallas as pl
from jax.experimental.pallas import tpu as pltpu
from jax.experimental.pallas import tpu_sc as plsc

_L = 16
_NW = 32


@functools.lru_cache(maxsize=None)
def _build(B, S, D):
    SP = -(-S // _L) * _L
    NCH = SP // _L
    RW = B // _NW
    segs = []
    off = 0
    while off < SP:
        n = min(128, SP - off)
        segs.append((off, n))
        off += n

    mesh = plsc.VectorSubcoreMesh(core_axis_name="c", subcore_axis_name="s")

    @functools.partial(
        pl.kernel,
        out_type=jax.ShapeDtypeStruct((B * S, D), jnp.float32),
        mesh=mesh,
        scratch_types=[
            pltpu.VMEM((RW, SP), jnp.int32),
            pltpu.VMEM((SP,), jnp.int32),
            pltpu.VMEM((SP, D), jnp.float32),
            pltpu.SemaphoreType.DMA,
        ],
        compiler_params=pltpu.CompilerParams(
            needs_layout_passes=False, use_tc_tiling_on_sc=False
        ),
    )
    def lor_kernel(xf, obs, out, obs_v, idx_v, rows_v, sem):
        wid = lax.axis_index("s") * 2 + lax.axis_index("c")
        row0 = wid * RW
        pltpu.sync_copy(obs.at[pl.ds(row0, RW)], obs_v)

        iota = lax.iota(jnp.int32, _L)
        shift = jnp.maximum(iota - 1, 0)

        def body(r, acc):
            bS = (row0 + r) * S
            carry = jnp.int32(0)
            for c in range(NCH):
                v = obs_v[r, pl.ds(c * _L, _L)]
                vals = v * (iota + (c * _L))
                inc = plsc.cummax(vals)
                sh = inc.at[shift].get(mode="promise_in_bounds")
                sh = jnp.where(iota == 0, 0, sh)
                ex = jnp.maximum(sh, carry)
                idx_v[pl.ds(c * _L, _L)] = ex + bS
                carry = jnp.maximum(carry, jnp.max(inc))
            copies = [
                pltpu.async_copy(
                    xf.at[idx_v.at[pl.ds(o, n)]], rows_v.at[pl.ds(o, n)], sem
                )
                for o, n in segs
            ]
            for cp in copies:
                cp.wait()
            pltpu.sync_copy(rows_v.at[pl.ds(0, S)], out.at[pl.ds(bS, S)])
            return acc

        lax.fori_loop(0, RW, body, 0)

    return lor_kernel


def kernel(x, observed):
    B, S, D = x.shape
    SP = -(-S // _L) * _L
    xf = x.reshape(B * S, D)
    obs = jnp.pad(observed.astype(jnp.int32), ((0, 0), (0, SP - S)))
    z = _build(B, S, D)(xf, obs)
    return z.reshape(B, S, D)


# speedup vs baseline: 7.8467x; 2.6569x over previous
import functools

import jax
import jax.numpy as jnp
from jax import lax
from jax.experimental import pallas as pl
from jax.experimental.pallas import tpu as pltpu
from jax.experimental.pallas import tpu_sc as plsc

_L = 16
_NW = 32
_BW = 128


@functools.lru_cache(maxsize=None)
def _build(B, S, D):
    NG = _BW // _L
    mesh = plsc.VectorSubcoreMesh(core_axis_name="c", subcore_axis_name="s")

    @functools.partial(
        pl.kernel,
        out_type=jax.ShapeDtypeStruct((S, D, B), jnp.float32),
        mesh=mesh,
        scratch_types=[
            pltpu.VMEM((S, _BW), jnp.int32),
            pltpu.VMEM((D, _BW), jnp.float32),
            pltpu.VMEM((D, _BW), jnp.float32),
            pltpu.VMEM((D, _BW), jnp.float32),
            pltpu.VMEM((D, _BW), jnp.float32),
            pltpu.SemaphoreType.DMA,
            pltpu.SemaphoreType.DMA,
            pltpu.SemaphoreType.DMA,
            pltpu.SemaphoreType.DMA,
            pltpu.SemaphoreType.DMA,
        ],
        compiler_params=pltpu.CompilerParams(needs_layout_passes=False),
    )
    def lor_kernel(xp, obs, out, obs_v, xb0, xb1, cb0, cb1,
                   sem_obs, sem_x0, sem_x1, sem_c0, sem_c1):
        wid = lax.axis_index("s") * 2 + lax.axis_index("c")
        b0 = wid * _BW
        xbs = (xb0, xb1)
        cbs = (cb0, cb1)
        sem_xs = (sem_x0, sem_x1)
        sem_cs = (sem_c0, sem_c1)

        def x_dma(s, buf, sem):
            return pltpu.make_async_copy(
                xp.at[s, :, pl.ds(b0, _BW)], buf, sem
            )

        def out_dma(buf, t, sem):
            return pltpu.make_async_copy(
                buf, out.at[t, :, pl.ds(b0, _BW)], sem
            )

        obs_cp = pltpu.make_async_copy(
            obs.at[:, pl.ds(b0, _BW)], obs_v, sem_obs
        )
        obs_cp.start()
        x_dma(0, cb0, sem_c0).start()
        x_dma(0, cb1, sem_c1).start()
        x_dma(1, xb1, sem_x1).start()
        x_dma(0, cb0, sem_c0).wait()
        out_dma(cb0, 0, sem_c0).start()
        x_dma(0, cb1, sem_c1).wait()
        out_dma(cb1, 1, sem_c1).start()
        obs_cp.wait()

        def update(t, j):
            s = t - 1
            x_dma(t, xbs[j], sem_xs[j]).start()
            x_dma(s, xbs[1 - j], sem_xs[1 - j]).wait()
            out_dma(cbs[j], t - 2, sem_cs[j]).wait()
            masks = [obs_v[s, pl.ds(k * _L, _L)] != 0 for k in range(NG)]
            xsrc = xbs[1 - j]
            csrc = cbs[1 - j]
            cdst = cbs[j]
            for d in range(D):
                for k in range(NG):
                    sl = pl.ds(k * _L, _L)
                    cdst[d, sl] = jnp.where(masks[k], xsrc[d, sl], csrc[d, sl])
            out_dma(cdst, t, sem_cs[j]).start()

        def body(i, acc):
            t0 = 2 * i + 2
            update(t0, 0)
            update(t0 + 1, 1)
            return acc

        lax.fori_loop(0, (S - 2) // 2, body, 0)

        out_dma(cb0, S - 2, sem_c0).wait()
        out_dma(cb1, S - 1, sem_c1).wait()
        x_dma(S - 1, xb1, sem_x1).wait()

    return lor_kernel


def kernel(x, observed):
    B, S, D = x.shape
    xp = jnp.transpose(x, (1, 2, 0))
    obsT = jnp.transpose(observed.astype(jnp.int32), (1, 0))
    outp = _build(B, S, D)(xp, obsT)
    return jnp.transpose(outp, (2, 0, 1))


# speedup vs baseline: 20.7671x; 2.6466x over previous
import functools

import jax
import jax.numpy as jnp
from jax import lax
from jax.experimental import pallas as pl
from jax.experimental.pallas import tpu as pltpu
from jax.experimental.pallas import tpu_sc as plsc

_L = 16
_NW = 32
_BW = 128


@functools.lru_cache(maxsize=None)
def _build(B, S, D):
    NG = _BW // _L
    NP = S // 2 - 1
    mesh = plsc.VectorSubcoreMesh(core_axis_name="c", subcore_axis_name="s")

    @functools.partial(
        pl.kernel,
        out_type=jax.ShapeDtypeStruct((S, D, B), jnp.float32),
        mesh=mesh,
        scratch_types=[
            pltpu.VMEM((S, _BW), jnp.int32),
            pltpu.VMEM((2, D, _BW), jnp.float32),
            pltpu.VMEM((2, D, _BW), jnp.float32),
            pltpu.VMEM((2, D, _BW), jnp.float32),
            pltpu.VMEM((2, D, _BW), jnp.float32),
            pltpu.SemaphoreType.DMA,
            pltpu.SemaphoreType.DMA,
            pltpu.SemaphoreType.DMA,
            pltpu.SemaphoreType.DMA,
            pltpu.SemaphoreType.DMA,
        ],
        compiler_params=pltpu.CompilerParams(needs_layout_passes=False),
    )
    def lor_kernel(xp, obs, out, obs_v, xb0, xb1, cb0, cb1,
                   sem_obs, sem_x0, sem_x1, sem_c0, sem_c1):
        wid = lax.axis_index("s") * 2 + lax.axis_index("c")
        b0 = wid * _BW
        xbs = (xb0, xb1)
        cbs = (cb0, cb1)
        sem_xs = (sem_x0, sem_x1)
        sem_cs = (sem_c0, sem_c1)

        def xpair_dma(p, q):
            s0 = jnp.minimum(2 * p + 1, S - 2)
            return pltpu.make_async_copy(
                xp.at[pl.ds(s0, 2), :, pl.ds(b0, _BW)], xbs[q], sem_xs[q]
            )

        def x0_dma(slot, q):
            return pltpu.make_async_copy(
                xp.at[pl.ds(0, 1), :, pl.ds(b0, _BW)],
                cbs[q].at[pl.ds(slot, 1)], sem_cs[q]
            )

        def out_dma(q, t0):
            return pltpu.make_async_copy(
                cbs[q], out.at[pl.ds(t0, 2), :, pl.ds(b0, _BW)], sem_cs[q]
            )

        obs_cp = pltpu.make_async_copy(
            obs.at[:, pl.ds(b0, _BW)], obs_v, sem_obs
        )
        obs_cp.start()
        x0_dma(0, 1).start()
        x0_dma(1, 1).start()
        xpair_dma(0, 0).start()
        x0_dma(0, 1).wait()
        x0_dma(1, 1).wait()
        out_dma(1, 0).start()
        obs_cp.wait()

        def do_pair(p, q, first=False):
            t0 = 2 * p + 2
            xpair_dma(p + 1, 1 - q).start()
            xpair_dma(p, q).wait()
            if not first:
                out_dma(q, t0 - 4).wait()
            xsrc = xbs[q]
            cprev = cbs[1 - q]
            cdst = cbs[q]
            m0 = [obs_v[t0 - 1, pl.ds(k * _L, _L)] != 0 for k in range(NG)]
            m1 = [obs_v[t0, pl.ds(k * _L, _L)] != 0 for k in range(NG)]

            @plsc.parallel_loop(0, D, step=1, unroll=8)
            def _upd0(d):
                for k in range(NG):
                    sl = pl.ds(k * _L, _L)
                    cdst[0, d, sl] = jnp.where(
                        m0[k], xsrc[0, d, sl], cprev[1, d, sl]
                    )

            @plsc.parallel_loop(0, D, step=1, unroll=8)
            def _upd1(d):
                for k in range(NG):
                    sl = pl.ds(k * _L, _L)
                    cdst[1, d, sl] = jnp.where(
                        m1[k], xsrc[1, d, sl], cdst[0, d, sl]
                    )

            out_dma(q, t0).start()

        def body(i, acc):
            do_pair(2 * i + 1, 1)
            do_pair(2 * i + 2, 0)
            return acc

        do_pair(0, 0, first=True)
        lax.fori_loop(0, (NP - 1) // 2, body, 0)

        out_dma((NP - 2) % 2, S - 4).wait()
        out_dma((NP - 1) % 2, S - 2).wait()
        xpair_dma(NP, NP % 2).wait()

    return lor_kernel


def kernel(x, observed):
    B, S, D = x.shape
    xp = jnp.transpose(x, (1, 2, 0))
    obsT = jnp.transpose(observed.astype(jnp.int32), (1, 0))
    outp = _build(B, S, D)(xp, obsT)
    return jnp.transpose(outp, (2, 0, 1))
